# baseline (speedup 1.0000x reference)
import functools

import jax
import jax.numpy as jnp
from jax import lax
from jax.experimental import pallas as pl
from jax.experimental.pallas import tpu as pltpu
from jax.experimental.pallas import tpu_sc as plsc

N_NODES = 10000
NPAD = 10240
N_EDGES = 320000
DIM = 128

NC, NS = 2, 16
NW = NC * NS
EPT = N_EDGES // NW
CHUNK = 80
NSTEPS = EPT // CHUNK
ROWS_PT = NPAD // NS


def _sc_agg_body(feats_hbm, src_hbm, dst_hbm, agg_out,
                 idx_sb, idx_d0, idx_d1, idx_d2,
                 rows0, rows1, rows2, acc_sh,
                 bsem, gsem0, gsem1, gsem2, ssem0, ssem1, ssem2,
                 isem0, isem1, isem2):
  cid = lax.axis_index("c")
  sid = lax.axis_index("s")
  wid = sid * NC + cid
  z16 = jnp.zeros((16,), jnp.float32)
  bufs = ((idx_d0, rows0, gsem0, ssem0, isem0),
          (idx_d1, rows1, gsem1, ssem1, isem1),
          (idx_d2, rows2, gsem2, ssem2, isem2))
  estart = wid * EPT

  big = pltpu.async_copy(src_hbm.at[pl.ds(estart, EPT)], idx_sb, bsem)

  def zb(i, _):
    r = i // (DIM // 16)
    c = (i % (DIM // 16)) * 16
    rows0[r, pl.ds(c, 16)] = z16
    return _
  lax.fori_loop(0, CHUNK * (DIM // 16), zb, None)

  rbase = sid * ROWS_PT
  for j in range(ROWS_PT // CHUNK):
    pltpu.sync_copy(rows0, acc_sh.at[pl.ds(rbase + j * CHUNK, CHUNK)])
  big.wait()
  plsc.subcore_barrier()

  def issue(i, b):
    idx_d, rows, gsem, _, isem = bufs[b]
    off = estart + i * CHUNK
    pltpu.async_copy(dst_hbm.at[pl.ds(off, CHUNK)], idx_d, isem)
    pltpu.async_copy(feats_hbm.at[idx_sb.at[pl.ds(i * CHUNK, CHUNK)]],
                     rows, gsem)

  def wait_gather(i, b):
    idx_d, rows, gsem, _, isem = bufs[b]
    pltpu.make_async_copy(feats_hbm.at[idx_sb.at[pl.ds(i * CHUNK, CHUNK)]],
                          rows, gsem).wait()
    off = estart + i * CHUNK
    pltpu.make_async_copy(dst_hbm.at[pl.ds(off, CHUNK)], idx_d, isem).wait()

  def wait_scatter(b):
    idx_d, rows, _, ssem, _ = bufs[b]
    pltpu.make_async_copy(rows, acc_sh.at[idx_d], ssem).wait()

  issue(0, 0)
  issue(1, 1)

  def tri(i3, _):
    for b in range(3):
      i = i3 * 3 + b
      @pl.when(i < NSTEPS)
      def _process():
        idx_d, rows, _, ssem, _ = bufs[b]
        wait_gather(i, b)
        pltpu.async_copy(rows, acc_sh.at[idx_d], ssem, add=True)
        @pl.when(i + 2 < NSTEPS)
        def _refill():
          b2 = (b + 2) % 3
          @pl.when(i >= 1)
          def _drain():
            wait_scatter(b2)
          issue(i + 2, b2)
    return _
  lax.fori_loop(0, (NSTEPS + 2) // 3, tri, None)

  wait_scatter((NSTEPS - 3) % 3)
  wait_scatter((NSTEPS - 2) % 3)
  wait_scatter((NSTEPS - 1) % 3)
  plsc.subcore_barrier()

  pltpu.sync_copy(acc_sh.at[pl.ds(rbase, ROWS_PT)],
                  agg_out.at[cid, pl.ds(rbase, ROWS_PT)])


def _sc_cnt_body(dst_hbm, cnt_out, idx_d0, idx_d1, idx_d2, ones, cnt_sh,
                 ssem0, ssem1, ssem2, isem0, isem1, isem2):
  cid = lax.axis_index("c")
  sid = lax.axis_index("s")
  wid = sid * NC + cid
  z16 = jnp.zeros((16,), jnp.float32)
  o16 = jnp.ones((16,), jnp.float32)
  bufs = ((idx_d0, ssem0, isem0), (idx_d1, ssem1, isem1),
          (idx_d2, ssem2, isem2))
  estart = wid * EPT

  def zb(i, _):
    r = i // (DIM // 16)
    c = (i % (DIM // 16)) * 16
    ones[r, pl.ds(c, 16)] = z16
    return _
  lax.fori_loop(0, CHUNK * (DIM // 16), zb, None)

  rbase = sid * ROWS_PT
  for j in range(ROWS_PT // CHUNK):
    pltpu.sync_copy(ones, cnt_sh.at[pl.ds(rbase + j * CHUNK, CHUNK)])

  def ob(i, _):
    r = i // (DIM // 16)
    c = (i % (DIM // 16)) * 16
    ones[r, pl.ds(c, 16)] = o16
    return _
  lax.fori_loop(0, CHUNK * (DIM // 16), ob, None)
  plsc.subcore_barrier()

  ones_rows = ones

  def issue(i, b):
    idx_d, _, isem = bufs[b]
    off = estart + i * CHUNK
    pltpu.async_copy(dst_hbm.at[pl.ds(off, CHUNK)], idx_d, isem)

  def wait_idx(i, b):
    idx_d, _, isem = bufs[b]
    off = estart + i * CHUNK
    pltpu.make_async_copy(dst_hbm.at[pl.ds(off, CHUNK)], idx_d, isem).wait()

  def wait_scatter(b):
    idx_d, ssem, _ = bufs[b]
    pltpu.make_async_copy(ones_rows, cnt_sh.at[idx_d], ssem).wait()

  issue(0, 0)
  issue(1, 1)

  def tri(i3, _):
    for b in range(3):
      i = i3 * 3 + b
      @pl.when(i < NSTEPS)
      def _process():
        idx_d, ssem, _ = bufs[b]
        wait_idx(i, b)
        pltpu.async_copy(ones_rows, cnt_sh.at[idx_d], ssem, add=True)
        @pl.when(i + 2 < NSTEPS)
        def _refill():
          b2 = (b + 2) % 3
          @pl.when(i >= 1)
          def _drain():
            wait_scatter(b2)
          issue(i + 2, b2)
    return _
  lax.fori_loop(0, (NSTEPS + 2) // 3, tri, None)

  wait_scatter((NSTEPS - 3) % 3)
  wait_scatter((NSTEPS - 2) % 3)
  wait_scatter((NSTEPS - 1) % 3)
  plsc.subcore_barrier()

  pltpu.sync_copy(cnt_sh.at[pl.ds(rbase, ROWS_PT)],
                  cnt_out.at[cid, pl.ds(rbase, ROWS_PT)])


_SC_MESH = plsc.VectorSubcoreMesh(core_axis_name="c", subcore_axis_name="s")

_sc_agg = pl.kernel(
    _sc_agg_body,
    out_type=jax.ShapeDtypeStruct((NC, NPAD, DIM), jnp.float32),
    mesh=_SC_MESH,
    scratch_types=(
        [pltpu.VMEM((EPT,), jnp.int32)] +
        [pltpu.VMEM((CHUNK,), jnp.int32)] * 3 +
        [pltpu.VMEM((CHUNK, DIM), jnp.float32)] * 3 +
        [pltpu.VMEM_SHARED((NPAD, DIM), jnp.float32)] +
        [pltpu.SemaphoreType.DMA] * 10
    ),
    name="sc_sage_aggregate",
)

_sc_cnt = pl.kernel(
    _sc_cnt_body,
    out_type=jax.ShapeDtypeStruct((NC, NPAD, DIM), jnp.float32),
    mesh=_SC_MESH,
    scratch_types=(
        [pltpu.VMEM((CHUNK,), jnp.int32)] * 3 +
        [pltpu.VMEM((CHUNK, DIM), jnp.float32),
         pltpu.VMEM_SHARED((NPAD, DIM), jnp.float32)] +
        [pltpu.SemaphoreType.DMA] * 6
    ),
    name="sc_sage_degree",
)

BR = 1000


def _tc_layer_body(do_relu, parts_ref, cnt_ref, x_ref, wl_ref, bl_ref,
                   wr_ref, o_ref):
  agg = parts_ref[0] + parts_ref[1]
  cnt = cnt_ref[0, :, 0:1] + cnt_ref[1, :, 0:1]
  mean = agg / jnp.maximum(cnt, 1.0)
  y = jnp.dot(mean, wl_ref[...], preferred_element_type=jnp.float32)
  y = y + bl_ref[...]
  y = y + jnp.dot(x_ref[...], wr_ref[...], preferred_element_type=jnp.float32)
  if do_relu:
    y = jnp.maximum(y, 0.0)
  o_ref[...] = y


def _tc_layer(parts, cnt, x, Wl, bl, Wr, do_relu):
  grid = (N_NODES // BR,)
  return pl.pallas_call(
      functools.partial(_tc_layer_body, do_relu),
      grid=grid,
      in_specs=[
          pl.BlockSpec((NC, BR, DIM), lambda i: (0, i, 0)),
          pl.BlockSpec((NC, BR, DIM), lambda i: (0, i, 0)),
          pl.BlockSpec((BR, DIM), lambda i: (i, 0)),
          pl.BlockSpec((DIM, DIM), lambda i: (0, 0)),
          pl.BlockSpec((1, DIM), lambda i: (0, 0)),
          pl.BlockSpec((DIM, DIM), lambda i: (0, 0)),
      ],
      out_specs=pl.BlockSpec((BR, DIM), lambda i: (i, 0)),
      out_shape=jax.ShapeDtypeStruct((N_NODES, DIM), jnp.float32),
      name="tc_sage_linear" + ("_relu" if do_relu else ""),
  )(parts, cnt, x, Wl, bl.reshape(1, DIM), Wr)


def kernel(x, edge_index, edge_weight, Wl1, bl1, Wr1, Wl2, bl2, Wr2):
  del edge_weight
  src = edge_index[0]
  dst = edge_index[1]
  cnt = _sc_cnt(dst)
  agg1 = _sc_agg(x, src, dst)
  h = _tc_layer(agg1, cnt, x, Wl1, bl1, Wr1, do_relu=True)
  agg2 = _sc_agg(h, src, dst)
  out = _tc_layer(agg2, cnt, h, Wl2, bl2, Wr2, do_relu=False)
  return out


# speedup vs baseline: 1.0109x; 1.0109x over previous
import functools

import jax
import jax.numpy as jnp
from jax import lax
from jax.experimental import pallas as pl
from jax.experimental.pallas import tpu as pltpu
from jax.experimental.pallas import tpu_sc as plsc

N_NODES = 10000
NPAD = 10240
N_EDGES = 320000
DIM = 128

NC, NS = 2, 16
NW = NC * NS
EPT = N_EDGES // NW
CHUNK = 80
NSTEPS = EPT // CHUNK
ROWS_PT = NPAD // NS


def _sc_agg_body(with_cnt, feats_hbm, src_hbm, dst_hbm, *rest):
  if with_cnt:
    (agg_out, cnt_out, idx_sb, idx_d0, idx_d1, idx_d2,
     rows0, rows1, rows2, acc_sh,
     bsem, gsem0, gsem1, gsem2, ssem0, ssem1, ssem2,
     isem0, isem1, isem2) = rest
  else:
    (agg_out, idx_sb, idx_d0, idx_d1, idx_d2,
     rows0, rows1, rows2, acc_sh,
     bsem, gsem0, gsem1, gsem2, ssem0, ssem1, ssem2,
     isem0, isem1, isem2) = rest
  cid = lax.axis_index("c")
  sid = lax.axis_index("s")
  wid = sid * NC + cid
  z16 = jnp.zeros((16,), jnp.float32)
  bufs = ((idx_d0, rows0, gsem0, ssem0, isem0),
          (idx_d1, rows1, gsem1, ssem1, isem1),
          (idx_d2, rows2, gsem2, ssem2, isem2))
  estart = wid * EPT

  big = pltpu.async_copy(src_hbm.at[pl.ds(estart, EPT)], idx_sb, bsem)

  def zb(i, _):
    r = i // (DIM // 16)
    c = (i % (DIM // 16)) * 16
    rows0[r, pl.ds(c, 16)] = z16
    return _
  lax.fori_loop(0, CHUNK * (DIM // 16), zb, None)

  rbase = sid * ROWS_PT
  for j in range(ROWS_PT // CHUNK):
    pltpu.sync_copy(rows0, acc_sh.at[pl.ds(rbase + j * CHUNK, CHUNK)])
  big.wait()
  plsc.subcore_barrier()

  if with_cnt:
    o16 = jnp.ones((16,), jnp.float32)
    def ob(i, _):
      r = i // (DIM // 16)
      c = (i % (DIM // 16)) * 16
      rows0[r, pl.ds(c, 16)] = o16
      return _
    lax.fori_loop(0, CHUNK * (DIM // 16), ob, None)

    def c_issue(i, b):
      idx_d = bufs[b][0]
      isem = bufs[b][4]
      off = estart + i * CHUNK
      pltpu.async_copy(dst_hbm.at[pl.ds(off, CHUNK)], idx_d, isem)

    def c_wait_idx(i, b):
      idx_d = bufs[b][0]
      isem = bufs[b][4]
      off = estart + i * CHUNK
      pltpu.make_async_copy(dst_hbm.at[pl.ds(off, CHUNK)], idx_d,
                            isem).wait()

    def c_wait_scatter(b):
      idx_d = bufs[b][0]
      ssem = bufs[b][3]
      pltpu.make_async_copy(rows0, acc_sh.at[idx_d], ssem).wait()

    c_issue(0, 0)
    c_issue(1, 1)

    def c_tri(i3, _):
      for b in range(3):
        i = i3 * 3 + b
        @pl.when(i < NSTEPS)
        def _process():
          idx_d = bufs[b][0]
          ssem = bufs[b][3]
          c_wait_idx(i, b)
          pltpu.async_copy(rows0, acc_sh.at[idx_d], ssem, add=True)
          @pl.when(i + 2 < NSTEPS)
          def _refill():
            b2 = (b + 2) % 3
            @pl.when(i >= 1)
            def _drain():
              c_wait_scatter(b2)
            c_issue(i + 2, b2)
      return _
    lax.fori_loop(0, (NSTEPS + 2) // 3, c_tri, None)

    c_wait_scatter((NSTEPS - 3) % 3)
    c_wait_scatter((NSTEPS - 2) % 3)
    c_wait_scatter((NSTEPS - 1) % 3)
    plsc.subcore_barrier()

    pltpu.sync_copy(acc_sh.at[pl.ds(rbase, ROWS_PT)],
                    cnt_out.at[cid, pl.ds(rbase, ROWS_PT)])
    def zb2(i, _):
      r = i // (DIM // 16)
      c = (i % (DIM // 16)) * 16
      rows0[r, pl.ds(c, 16)] = z16
      return _
    lax.fori_loop(0, CHUNK * (DIM // 16), zb2, None)
    for j in range(ROWS_PT // CHUNK):
      pltpu.sync_copy(rows0, acc_sh.at[pl.ds(rbase + j * CHUNK, CHUNK)])
    plsc.subcore_barrier()

  def issue(i, b):
    idx_d, rows, gsem, _, isem = bufs[b]
    off = estart + i * CHUNK
    pltpu.async_copy(dst_hbm.at[pl.ds(off, CHUNK)], idx_d, isem)
    pltpu.async_copy(feats_hbm.at[idx_sb.at[pl.ds(i * CHUNK, CHUNK)]],
                     rows, gsem)

  def wait_gather(i, b):
    idx_d, rows, gsem, _, isem = bufs[b]
    pltpu.make_async_copy(feats_hbm.at[idx_sb.at[pl.ds(i * CHUNK, CHUNK)]],
                          rows, gsem).wait()
    off = estart + i * CHUNK
    pltpu.make_async_copy(dst_hbm.at[pl.ds(off, CHUNK)], idx_d, isem).wait()

  def wait_scatter(b):
    idx_d, rows, _, ssem, _ = bufs[b]
    pltpu.make_async_copy(rows, acc_sh.at[idx_d], ssem).wait()

  issue(0, 0)
  issue(1, 1)

  def tri(i3, _):
    for b in range(3):
      i = i3 * 3 + b
      @pl.when(i < NSTEPS)
      def _process():
        idx_d, rows, _, ssem, _ = bufs[b]
        wait_gather(i, b)
        pltpu.async_copy(rows, acc_sh.at[idx_d], ssem, add=True)
        @pl.when(i + 2 < NSTEPS)
        def _refill():
          b2 = (b + 2) % 3
          @pl.when(i >= 1)
          def _drain():
            wait_scatter(b2)
          issue(i + 2, b2)
    return _
  lax.fori_loop(0, (NSTEPS + 2) // 3, tri, None)

  wait_scatter((NSTEPS - 3) % 3)
  wait_scatter((NSTEPS - 2) % 3)
  wait_scatter((NSTEPS - 1) % 3)
  plsc.subcore_barrier()

  pltpu.sync_copy(acc_sh.at[pl.ds(rbase, ROWS_PT)],
                  agg_out.at[cid, pl.ds(rbase, ROWS_PT)])


_SC_MESH = plsc.VectorSubcoreMesh(core_axis_name="c", subcore_axis_name="s")

def _make_sc_agg(with_cnt):
  outs = [jax.ShapeDtypeStruct((NC, NPAD, DIM), jnp.float32)]
  if with_cnt:
    outs = outs + [jax.ShapeDtypeStruct((NC, NPAD, DIM), jnp.float32)]
  return pl.kernel(
      functools.partial(_sc_agg_body, with_cnt),
      out_type=tuple(outs) if with_cnt else outs[0],
      mesh=_SC_MESH,
      scratch_types=(
          [pltpu.VMEM((EPT,), jnp.int32)] +
          [pltpu.VMEM((CHUNK,), jnp.int32)] * 3 +
          [pltpu.VMEM((CHUNK, DIM), jnp.float32)] * 3 +
          [pltpu.VMEM_SHARED((NPAD, DIM), jnp.float32)] +
          [pltpu.SemaphoreType.DMA] * 10
      ),
      name="sc_sage_aggregate" + ("_cnt" if with_cnt else ""),
  )


_sc_agg_cnt = _make_sc_agg(True)
_sc_agg = _make_sc_agg(False)

BR = 1000


def _tc_layer_body(do_relu, parts_ref, cnt_ref, x_ref, wl_ref, bl_ref,
                   wr_ref, o_ref):
  agg = parts_ref[0] + parts_ref[1]
  cnt = cnt_ref[0, :, 0:1] + cnt_ref[1, :, 0:1]
  mean = agg / jnp.maximum(cnt, 1.0)
  y = jnp.dot(mean, wl_ref[...], preferred_element_type=jnp.float32)
  y = y + bl_ref[...]
  y = y + jnp.dot(x_ref[...], wr_ref[...], preferred_element_type=jnp.float32)
  if do_relu:
    y = jnp.maximum(y, 0.0)
  o_ref[...] = y


def _tc_layer(parts, cnt, x, Wl, bl, Wr, do_relu):
  grid = (N_NODES // BR,)
  return pl.pallas_call(
      functools.partial(_tc_layer_body, do_relu),
      grid=grid,
      in_specs=[
          pl.BlockSpec((NC, BR, DIM), lambda i: (0, i, 0)),
          pl.BlockSpec((NC, BR, DIM), lambda i: (0, i, 0)),
          pl.BlockSpec((BR, DIM), lambda i: (i, 0)),
          pl.BlockSpec((DIM, DIM), lambda i: (0, 0)),
          pl.BlockSpec((1, DIM), lambda i: (0, 0)),
          pl.BlockSpec((DIM, DIM), lambda i: (0, 0)),
      ],
      out_specs=pl.BlockSpec((BR, DIM), lambda i: (i, 0)),
      out_shape=jax.ShapeDtypeStruct((N_NODES, DIM), jnp.float32),
      name="tc_sage_linear" + ("_relu" if do_relu else ""),
  )(parts, cnt, x, Wl, bl.reshape(1, DIM), Wr)


def kernel(x, edge_index, edge_weight, Wl1, bl1, Wr1, Wl2, bl2, Wr2):
  del edge_weight
  src = edge_index[0]
  dst = edge_index[1]
  agg1, cnt = _sc_agg_cnt(x, src, dst)
  h = _tc_layer(agg1, cnt, x, Wl1, bl1, Wr1, do_relu=True)
  agg2 = _sc_agg(h, src, dst)
  out = _tc_layer(agg2, cnt, h, Wl2, bl2, Wr2, do_relu=False)
  return out
